# baseline (speedup 1.0000x reference)
import functools

import jax
import jax.numpy as jnp
from jax import lax
from jax.experimental import pallas as pl
from jax.experimental.pallas import tpu as pltpu
from jax.experimental.pallas import tpu_sc as plsc

_NC, _NS, _L = 2, 16, 16
_NW = _NC * _NS


def _make_sc_kernel(rows, enc):
    rpw = rows // _NW
    tmpl_rows = 8
    n_fill = rpw // tmpl_rows
    n_scat = rpw // 128
    mesh = plsc.VectorSubcoreMesh(core_axis_name="c", subcore_axis_name="s")

    @functools.partial(
        pl.kernel,
        out_type=jax.ShapeDtypeStruct((rows * enc,), jnp.float32),
        mesh=mesh,
        scratch_types=[
            pltpu.VMEM((tmpl_rows * enc,), jnp.float32),
            pltpu.VMEM((rpw,), jnp.int32),
            pltpu.VMEM((n_scat, 128), jnp.int32),
            pltpu.VMEM((128,), jnp.float32),
            pltpu.SemaphoreType.DMA,
        ],
    )
    def sc_kernel(idx_hbm, out_hbm, tmpl_v, idx_v, pos_v, ones_v, sem):
        wid = lax.axis_index("c") * _NS + lax.axis_index("s")
        row_base = wid * rpw

        ninf = jnp.full((_L,), -jnp.inf, jnp.float32)

        def fill(i, carry):
            base = i * (_L * 8)
            for u in range(8):
                tmpl_v[pl.ds(base + u * _L, _L)] = ninf
            return carry

        lax.fori_loop(0, (tmpl_rows * enc) // (_L * 8), fill, 0)

        one = jnp.full((_L,), 1.0, jnp.float32)
        for u in range(128 // _L):
            ones_v[pl.ds(u * _L, _L)] = one

        pltpu.sync_copy(idx_hbm.at[pl.ds(row_base, rpw)], idx_v)
        for g in range(rpw // _L):
            col = idx_v[pl.ds(g * _L, _L)]
            row = lax.iota(jnp.int32, _L) + (row_base + g * _L)
            pos = row * enc + col
            pos_v[g * _L // 128, pl.ds((g * _L) % 128, _L)] = pos

        copies = []
        for k in range(n_fill):
            dst = out_hbm.at[pl.ds((row_base + k * tmpl_rows) * enc, tmpl_rows * enc)]
            copies.append(pltpu.async_copy(tmpl_v, dst, sem))
        for c in copies:
            c.wait()

        for j in range(n_scat):
            pltpu.async_copy(ones_v, out_hbm.at[pos_v.at[j]], sem).wait()

    return sc_kernel


def kernel(decoder_states, encoder_states, step, input_lengths, provided_attention):
    B, dec_seqlen = provided_attention.shape
    enc_seqlen = encoder_states.shape[1]
    rows = B * dec_seqlen
    idx = jnp.asarray(provided_attention, jnp.int32).reshape(rows)
    out = _make_sc_kernel(rows, enc_seqlen)(idx)
    return out.reshape(B, dec_seqlen, enc_seqlen)


# speedup vs baseline: 2.6919x; 2.6919x over previous
import functools

import jax
import jax.numpy as jnp
from jax import lax
from jax.experimental import pallas as pl
from jax.experimental.pallas import tpu as pltpu
from jax.experimental.pallas import tpu_sc as plsc

_NC, _NS, _L = 2, 16, 16
_NW = _NC * _NS
_C = 16


def _log2(n):
    b = n.bit_length() - 1
    assert (1 << b) == n
    return b


def _make_sc_kernel(B, dec, enc):
    rows = B * dec
    rpw = rows // _NW
    n_chunks = rpw // _C
    dec_shift, dec_mask = _log2(dec), dec - 1
    mesh = plsc.VectorSubcoreMesh(core_axis_name="c", subcore_axis_name="s")

    @functools.partial(
        pl.kernel,
        out_type=jax.ShapeDtypeStruct((B, dec, enc), jnp.float32),
        mesh=mesh,
        compiler_params=pltpu.CompilerParams(needs_layout_passes=False),
        scratch_types=[
            pltpu.VMEM((_C, enc), jnp.float32),
            pltpu.VMEM((_C, enc), jnp.float32),
            pltpu.VMEM((rpw,), jnp.int32),
            pltpu.SemaphoreType.DMA,
            pltpu.SemaphoreType.DMA,
        ],
    )
    def sc_kernel(idx_hbm, out_hbm, tmpl_a, tmpl_b, idx_v, sem_a, sem_b):
        wid = lax.axis_index("c") * _NS + lax.axis_index("s")
        row_base = wid * rpw
        b = lax.shift_right_logical(row_base, dec_shift)
        d0 = pl.multiple_of(lax.bitwise_and(row_base, dec_mask), rpw)

        ninf = jnp.full((_L,), -jnp.inf, jnp.float32)
        one = jnp.full((_L,), 1.0, jnp.float32)
        rowiota = lax.iota(jnp.int32, _L)

        def fill(r, carry):
            for t in (tmpl_a, tmpl_b):
                for u in range(enc // _L):
                    t[r, pl.ds(u * _L, _L)] = ninf
            return carry

        lax.fori_loop(0, _C, fill, 0)

        pltpu.sync_copy(idx_hbm.at[pl.ds(pl.multiple_of(row_base, rpw), rpw)], idx_v)

        bufs = (tmpl_a, tmpl_b)
        sems = (sem_a, sem_b)
        copies = [None] * n_chunks
        for k in range(n_chunks):
            t, sem = bufs[k % 2], sems[k % 2]
            if k >= 2:
                copies[k - 2].wait()
                oldcol = idx_v[pl.ds((k - 2) * _C, _L)]
                plsc.store_scatter(t, [rowiota, oldcol], ninf)
            col = idx_v[pl.ds(k * _C, _L)]
            plsc.store_scatter(t, [rowiota, col], one)
            dst = out_hbm.at[b, pl.ds(d0 + k * _C, _C)]
            copies[k] = pltpu.async_copy(t, dst, sem)
        copies[n_chunks - 2].wait()
        copies[n_chunks - 1].wait()

    return sc_kernel


def kernel(decoder_states, encoder_states, step, input_lengths, provided_attention):
    B, dec_seqlen = provided_attention.shape
    enc_seqlen = encoder_states.shape[1]
    idx = jnp.asarray(provided_attention, jnp.int32).reshape(B * dec_seqlen)
    return _make_sc_kernel(B, dec_seqlen, enc_seqlen)(idx)
